# baseline (speedup 1.0000x reference)
import functools

import jax
import jax.numpy as jnp
from jax import lax
from jax.experimental import pallas as pl
from jax.experimental.pallas import tpu as pltpu
from jax.experimental.pallas import tpu_sc as plsc

_N = 16
_NA = 3
_NO = 85
_H = 64
_W = 64
_HW = _H * _W
_C = _NA * _NO
_S = 256
_NCH = _HW // _S
_STRIDE = 8.0
_L = 16


def _decode_body(p_hbm, anch_hbm, out_hbm, in_v, out_v, anch_v):
    cid = lax.axis_index("c")
    sid = lax.axis_index("s")
    wid = sid * 2 + cid
    n = wid >> 1
    half = wid & 1

    pltpu.sync_copy(anch_hbm, anch_v)
    iota = lax.broadcasted_iota(jnp.int32, (_L,), 0)

    for a in range(_NA):
        m0 = anch_v[pl.ds(a * 2 * _L, _L)]
        m1 = anch_v[pl.ds((a * 2 + 1) * _L, _L)]
        prow = n * _C + a * _NO

        def chunk_body(ci, carry, a=a, m0=m0, m1=m1, prow=prow):
            c = half * (_NCH // 2) + ci
            orow = n * (_NA * _HW) + a * _HW + c * _S
            pltpu.sync_copy(
                p_hbm.at[pl.ds(prow, _NO), pl.ds(c * _S, _S)], in_v)

            for j in range(_S // _L):
                col = iota + j * _L
                obase = col * _NO
                x0 = in_v[0, pl.ds(j * _L, _L)]
                sg0 = 1.0 / (1.0 + jnp.exp(-x0))
                gx8 = ((col & (_W - 1)) * int(_STRIDE)).astype(jnp.float32)
                plsc.store_scatter(out_v, [obase], sg0 * _STRIDE + gx8)
                x1 = in_v[1, pl.ds(j * _L, _L)]
                sg1 = 1.0 / (1.0 + jnp.exp(-x1))
                gy8 = ((col >> 6) * int(_STRIDE)
                       + jnp.broadcast_to(c * (_S // _W) * int(_STRIDE),
                                          (_L,))).astype(jnp.float32)
                plsc.store_scatter(out_v, [obase + 1], sg1 * _STRIDE + gy8)
                x2 = in_v[2, pl.ds(j * _L, _L)]
                plsc.store_scatter(out_v, [obase + 2], jnp.exp(x2) * m0)
                x3 = in_v[3, pl.ds(j * _L, _L)]
                plsc.store_scatter(out_v, [obase + 3], jnp.exp(x3) * m1)

            def row_body(k, rc):
                kb = jnp.broadcast_to(k, (_L,))
                for j in range(_S // _L):
                    x = in_v[k, pl.ds(j * _L, _L)]
                    sg = 1.0 / (1.0 + jnp.exp(-x))
                    plsc.store_scatter(
                        out_v, [(iota + j * _L) * _NO + kb], sg)
                return rc

            lax.fori_loop(4, _NO, row_body, 0)

            pltpu.sync_copy(out_v, out_hbm.at[pl.ds(orow * _NO, _S * _NO)])
            return carry

        lax.fori_loop(0, _NCH // 2, chunk_body, 0)


def _sc_decode(p2, anchb):
    mesh = plsc.VectorSubcoreMesh(core_axis_name="c", subcore_axis_name="s")
    f = functools.partial(
        pl.kernel,
        mesh=mesh,
        out_type=jax.ShapeDtypeStruct((_N * _NA * _HW * _NO,), jnp.float32),
        scratch_types=[
            pltpu.VMEM((_NO, _S), jnp.float32),
            pltpu.VMEM((_S * _NO,), jnp.float32),
            pltpu.VMEM((_NA * 2 * _L,), jnp.float32),
        ],
        compiler_params=pltpu.CompilerParams(
            use_tc_tiling_on_sc=False, needs_layout_passes=False),
    )(_decode_body)
    return f(p2, anchb)


def kernel(preds, anchors):
    n, ch, h, w = preds.shape
    p2 = preds.reshape(n * ch, h * w)
    anchb = jnp.broadcast_to(
        anchors.reshape(_NA, 2, 1), (_NA, 2, _L)).reshape(_NA * 2 * _L)
    out1 = _sc_decode(p2, anchb.astype(jnp.float32))
    return out1.reshape(n, _NA * h * w, _NO)


# speedup vs baseline: 1.8794x; 1.8794x over previous
import functools

import jax
import jax.numpy as jnp
from jax import lax
from jax.experimental import pallas as pl
from jax.experimental.pallas import tpu as pltpu
from jax.experimental.pallas import tpu_sc as plsc

_N = 16
_NA = 3
_NO = 85
_H = 64
_W = 64
_HW = _H * _W
_C = _NA * _NO
_S = 256
_NCH = _HW // _S
_STRIDE = 8.0
_L = 16


def _decode_body(p_hbm, anch_hbm, out_hbm, in_v, out_v, anch_v):
    cid = lax.axis_index("c")
    sid = lax.axis_index("s")
    wid = sid * 2 + cid
    n = wid >> 1
    half = wid & 1

    pltpu.sync_copy(anch_hbm, anch_v)
    iota = lax.broadcasted_iota(jnp.int32, (_L,), 0)

    for a in range(_NA):
        m0 = anch_v[pl.ds(a * 2 * _L, _L)]
        m1 = anch_v[pl.ds((a * 2 + 1) * _L, _L)]
        prow = n * _C + a * _NO

        def chunk_body(ci, carry, a=a, m0=m0, m1=m1, prow=prow):
            c = half * (_NCH // 2) + ci
            orow = n * (_NA * _HW) + a * _HW + c * _S
            pltpu.sync_copy(
                p_hbm.at[pl.ds(prow, _NO), pl.ds(c * _S, _S)], in_v)

            for j in range(_S // _L):
                col = iota + j * _L
                obase = col * _NO
                x0 = in_v[0, pl.ds(j * _L, _L)]
                sg0 = 1.0 / (1.0 + jnp.exp(-x0))
                gx8 = ((col & (_W - 1)) * int(_STRIDE)).astype(jnp.float32)
                plsc.store_scatter(out_v, [obase], sg0 * _STRIDE + gx8)
                x1 = in_v[1, pl.ds(j * _L, _L)]
                sg1 = 1.0 / (1.0 + jnp.exp(-x1))
                gy8 = ((col >> 6) * int(_STRIDE)
                       + jnp.broadcast_to(c * (_S // _W) * int(_STRIDE),
                                          (_L,))).astype(jnp.float32)
                plsc.store_scatter(out_v, [obase + 1], sg1 * _STRIDE + gy8)
                x2 = in_v[2, pl.ds(j * _L, _L)]
                plsc.store_scatter(out_v, [obase + 2], jnp.exp(x2) * m0)
                x3 = in_v[3, pl.ds(j * _L, _L)]
                plsc.store_scatter(out_v, [obase + 3], jnp.exp(x3) * m1)

            @plsc.parallel_loop(4, _NO, unroll=3)
            def row_body(k):
                kb = jnp.broadcast_to(k, (_L,))
                for j in range(_S // _L):
                    x = in_v[k, pl.ds(j * _L, _L)]
                    sg = 1.0 / (1.0 + jnp.exp(-x))
                    plsc.store_scatter(
                        out_v, [(iota + j * _L) * _NO + kb], sg)

            pltpu.sync_copy(out_v, out_hbm.at[pl.ds(orow * _NO, _S * _NO)])
            return carry

        lax.fori_loop(0, _NCH // 2, chunk_body, 0)


def _sc_decode(p2, anchb):
    mesh = plsc.VectorSubcoreMesh(core_axis_name="c", subcore_axis_name="s")
    f = functools.partial(
        pl.kernel,
        mesh=mesh,
        out_type=jax.ShapeDtypeStruct((_N * _NA * _HW * _NO,), jnp.float32),
        scratch_types=[
            pltpu.VMEM((_NO, _S), jnp.float32),
            pltpu.VMEM((_S * _NO,), jnp.float32),
            pltpu.VMEM((_NA * 2 * _L,), jnp.float32),
        ],
        compiler_params=pltpu.CompilerParams(
            use_tc_tiling_on_sc=False, needs_layout_passes=False),
    )(_decode_body)
    return f(p2, anchb)


def kernel(preds, anchors):
    n, ch, h, w = preds.shape
    p2 = preds.reshape(n * ch, h * w)
    anchb = jnp.broadcast_to(
        anchors.reshape(_NA, 2, 1), (_NA, 2, _L)).reshape(_NA * 2 * _L)
    out1 = _sc_decode(p2, anchb.astype(jnp.float32))
    return out1.reshape(n, _NA * h * w, _NO)
